# Initial kernel scaffold; baseline (speedup 1.0000x reference)
import jax
import jax.numpy as jnp
from jax.experimental import pallas as pl


def kernel(x, edge_index, W_neigh0, W_self0, b0, W_neigh1, W_self1, b1, W_neigh2, W_self2, b2):
    raise NotImplementedError("write your pallas kernel here")


# speedup vs baseline: 4.5001x; 4.5001x over previous
import functools

import jax
import jax.numpy as jnp
from jax import lax
from jax.experimental import pallas as pl
from jax.experimental.pallas import tpu as pltpu
from jax.experimental.pallas import tpu_sc as plsc

N = 10000
E = 320000
NC = 2
NS = 16
NW = NC * NS
EPW = E // NW
CH = 80
NCHUNK = EPW // CH
WT = 10
RPW = N // WT
ZR = 200
NZ = RPW // ZR
NP_DEG = 10240
DPW = NP_DEG // NS

BN = 400
NB = N // BN


def _sc_agg(table, src, dst, with_deg):
    W = table.shape[1]
    mesh = plsc.VectorSubcoreMesh(core_axis_name="c", subcore_axis_name="s",
                                  num_cores=NC, num_subcores=NS)
    out_type = [jax.ShapeDtypeStruct((NC, N, W), jnp.float32)]
    if with_deg:
        out_type.append(jax.ShapeDtypeStruct((NC, NP_DEG), jnp.float32))
    scratch = [
        pltpu.VMEM((CH,), jnp.int32),
        pltpu.VMEM((CH,), jnp.int32),
        pltpu.VMEM((CH, W), jnp.float32),
        pltpu.VMEM((ZR, W), jnp.float32),
        pltpu.VMEM((CH,), jnp.float32),
        pltpu.VMEM((DPW,), jnp.float32),
        pltpu.VMEM_SHARED((N, W), jnp.float32),
        pltpu.VMEM_SHARED((NP_DEG,), jnp.float32),
        pltpu.SemaphoreType.DMA,
    ]

    def body(table_h, src_h, dst_h, acc_h, *rest):
        if with_deg:
            deg_h = rest[0]
            rest = rest[1:]
        i_src, i_dst, rows, zbuf, ones_v, dbuf, acc_sh, deg_sh, sem = rest
        c = lax.axis_index("c")
        s = lax.axis_index("s")
        wid = c * NS + s
        zero16 = jnp.zeros((16,), jnp.float32)
        per_row = W // 16

        def zb(i, _):
            zbuf[i // per_row, pl.ds((i % per_row) * 16, 16)] = zero16
            return 0
        lax.fori_loop(0, ZR * per_row, zb, 0)

        def zd(i, _):
            dbuf[pl.ds(i * 16, 16)] = zero16
            return 0
        lax.fori_loop(0, DPW // 16, zd, 0)

        def of(i, _):
            ones_v[pl.ds(i * 16, 16)] = jnp.ones((16,), jnp.float32)
            return 0
        lax.fori_loop(0, CH // 16, of, 0)

        @pl.when(s < WT)
        def _clear():
            for k in range(NZ):
                pltpu.sync_copy(zbuf, acc_sh.at[pl.ds(s * RPW + k * ZR, ZR)])
        if with_deg:
            pltpu.sync_copy(dbuf, deg_sh.at[pl.ds(s * DPW, DPW)])
        plsc.subcore_barrier()

        base0 = wid * EPW

        def chunk(i, _):
            b = base0 + i * CH
            pltpu.sync_copy(src_h.at[pl.ds(b, CH)], i_src)
            pltpu.sync_copy(dst_h.at[pl.ds(b, CH)], i_dst)
            pltpu.async_copy(table_h.at[i_src], rows, sem).wait()
            pltpu.sync_copy(rows, acc_sh.at[i_dst], add=True)
            if with_deg:
                pltpu.sync_copy(ones_v, deg_sh.at[i_dst], add=True)
            return 0
        lax.fori_loop(0, NCHUNK, chunk, 0)
        plsc.subcore_barrier()

        @pl.when(s < WT)
        def _writeback():
            for k in range(NZ):
                off = s * RPW + k * ZR
                pltpu.sync_copy(acc_sh.at[pl.ds(off, ZR)], zbuf)
                pltpu.sync_copy(zbuf, acc_h.at[c, pl.ds(off, ZR)])
        if with_deg:
            pltpu.sync_copy(deg_sh.at[pl.ds(s * DPW, DPW)], dbuf)
            pltpu.sync_copy(dbuf, deg_h.at[c, pl.ds(s * DPW, DPW)])

    res = pl.kernel(
        body,
        out_type=out_type,
        mesh=mesh,
        scratch_types=scratch,
        name=f"sc_agg_w{W}" + ("_deg" if with_deg else ""),
    )(table, src, dst)
    return res if with_deg else res[0]


def _tc_layer(deg3, h_prev, acc, wn, ws, b, relu):
    H = wn.shape[0]

    def body(d_ref, x_ref, a_ref, wn_ref, ws_ref, b_ref, o_ref):
        d = d_ref[0, 0, :] + d_ref[0, 1, :]
        recip = 1.0 / jnp.maximum(d, 1.0)
        am = (a_ref[0] + a_ref[1]) * recip[:, None]
        hn = lax.dot_general(am, wn_ref[...], (((1,), (1,)), ((), ())),
                             preferred_element_type=jnp.float32)
        hs = lax.dot_general(x_ref[...], ws_ref[...], (((1,), (1,)), ((), ())),
                             preferred_element_type=jnp.float32)
        o = hs + hn + b_ref[...]
        o_ref[...] = jnp.maximum(o, 0.0) if relu else o

    return pl.pallas_call(
        body,
        grid=(NB,),
        in_specs=[
            pl.BlockSpec((1, NC, BN), lambda i: (i, 0, 0)),
            pl.BlockSpec((BN, h_prev.shape[1]), lambda i: (i, 0)),
            pl.BlockSpec((NC, BN, acc.shape[2]), lambda i: (0, i, 0)),
            pl.BlockSpec(wn.shape, lambda i: (0, 0)),
            pl.BlockSpec(ws.shape, lambda i: (0, 0)),
            pl.BlockSpec((1, H), lambda i: (0, 0)),
        ],
        out_specs=pl.BlockSpec((BN, H), lambda i: (i, 0)),
        out_shape=jax.ShapeDtypeStruct((N, H), jnp.float32),
    )(deg3, h_prev, acc, wn, ws, b.reshape(1, H))


def kernel(x, edge_index, W_neigh0, W_self0, b0, W_neigh1, W_self1, b1,
           W_neigh2, W_self2, b2):
    src = edge_index[0].astype(jnp.int32)
    dst = edge_index[1].astype(jnp.int32)

    acc0, degp = _sc_agg(x, src, dst, with_deg=True)
    deg3 = degp[:, :N].reshape(NC, NB, BN).transpose(1, 0, 2)

    h0 = _tc_layer(deg3, x, acc0, W_neigh0, W_self0, b0, relu=True)
    acc1 = _sc_agg(h0, src, dst, with_deg=False)
    h1 = _tc_layer(deg3, h0, acc1, W_neigh1, W_self1, b1, relu=True)
    acc2 = _sc_agg(h1, src, dst, with_deg=False)
    return _tc_layer(deg3, h1, acc2, W_neigh2, W_self2, b2, relu=False)
